# baseline (speedup 1.0000x reference)
import functools

import jax
import jax.numpy as jnp
from jax import lax
from jax.experimental import pallas as pl
from jax.experimental.pallas import tpu as pltpu
from jax.experimental.pallas import tpu_sc as plsc

B = 64
N = 4096
D = 64
NC = 2
NS = 16
NW = NC * NS
L = 16
NV = N // L
DBLK = 4
NBLK = D // DBLK
NGLB = 2 * NBLK


def _sc_body(xt, out, ka0, kb0, ia0, ib0, h0, ka1, kb1, ia1, ib1, h1,
             xtile0, xtile1, otile, sem, semw):
    lane = lax.iota(jnp.int32, L)
    ones = jnp.full((L,), 1, jnp.int32)
    lanNV = lane * NV
    wid = lax.axis_index("s") * NC + lax.axis_index("c")
    b0 = wid * 2
    xtiles = (xtile0, xtile1)

    cpk0 = pltpu.async_copy(xt.at[b0 * D], otile.at[pl.ds(0, N)], sem)
    cpk1 = pltpu.async_copy(xt.at[(b0 + 1) * D], otile.at[pl.ds(N, N)], sem)

    def fetch(g):
        r, blk = g // NBLK, g % NBLK
        row0 = (b0 + r) * D + blk * DBLK
        xtile = xtiles[g % 2]
        return [pltpu.async_copy(xt.at[row0 + d],
                                 xtile.at[pl.ds(d * N, N)], sem)
                for d in range(DBLK)]

    pend_f = [fetch(0), fetch(1)]
    cpk0.wait()
    cpk1.wait()

    scope_sort = jax.named_scope("radix_sort")
    scope_sort.__enter__()

    @plsc.parallel_loop(0, NV, unroll=4)
    def init_body(i):
        for r, ka in ((0, ka0), (1, ka1)):
            v = plsc.bitcast(otile[pl.ds(r * N + i * L, L)], jnp.int32)
            ka[pl.ds(i * L, L)] = jnp.where(v < 0, ~v,
                                            v ^ jnp.int32(-2147483648))

    for p in range(4):
        if p % 2 == 0:
            pairs = ((ka0, kb0, ia0, ib0, h0), (ka1, kb1, ia1, ib1, h1))
        else:
            pairs = ((kb0, ka0, ib0, ia0, h0), (kb1, ka1, ib1, ia1, h1))
        shift = jnp.int32(8 * p)

        @plsc.parallel_loop(0, NV, unroll=4)
        def zero_body(i):
            h0[pl.ds(i * L, L)] = jnp.zeros((L,), jnp.int32)
            h1[pl.ds(i * L, L)] = jnp.zeros((L,), jnp.int32)

        @plsc.parallel_loop(0, NV, unroll=4)
        def count_body(i):
            pos = lanNV + i
            for (ks, _, _, _, h) in pairs:
                k = plsc.load_gather(ks, [pos])
                d = lax.shift_right_logical(k, shift) & 255
                plsc.addupdate_scatter(h, [d * L + lane], ones)

        def scan_body(i, c):
            c0, c1 = c
            v0 = h0[pl.ds(i * L, L)]
            inc0 = plsc.cumsum(v0)
            h0[pl.ds(i * L, L)] = inc0 - v0 + c0
            v1 = h1[pl.ds(i * L, L)]
            inc1 = plsc.cumsum(v1)
            h1[pl.ds(i * L, L)] = inc1 - v1 + c1
            return (c0 + inc0[15], c1 + inc1[15])

        lax.fori_loop(0, NV, scan_body, (jnp.int32(0), jnp.int32(0)),
                      unroll=4)

        def perm_body(i, _):
            pos = lanNV + i
            for (ks, kd, is_, id_, h) in pairs:
                k = plsc.load_gather(ks, [pos])
                if p == 0:
                    v = pos
                else:
                    v = plsc.load_gather(is_, [pos])
                d = lax.shift_right_logical(k, shift) & 255
                hi = d * L + lane
                dst = plsc.load_gather(h, [hi])
                if p != 3:
                    plsc.store_scatter(kd, [dst], k)
                plsc.store_scatter(id_, [dst], v)
                plsc.addupdate_scatter(h, [hi], ones)
            return 0

        lax.fori_loop(0, NV, perm_body, 0, unroll=4)

    scope_sort.__exit__(None, None, None)

    scope_perm = jax.named_scope("permute_stream")
    scope_perm.__enter__()
    pend_w = []
    for g in range(NGLB):
        r, blk = g // NBLK, g % NBLK
        row0 = (b0 + r) * D + blk * DBLK
        xtile = xtiles[g % 2]
        ia = ia0 if r == 0 else ia1
        for cp in pend_f[0]:
            cp.wait()
        pend_f = pend_f[1:]
        for cp in pend_w:
            cp.wait()

        @plsc.parallel_loop(0, NV, unroll=4)
        def permute_body(i):
            ordv = ia[pl.ds(i * L, L)]
            for d in range(DBLK):
                v = plsc.load_gather(xtile.at[pl.ds(d * N, N)], [ordv])
                otile[pl.ds(d * N + i * L, L)] = v

        if g + 2 < NGLB:
            pend_f.append(fetch(g + 2))
        pend_w = [pltpu.async_copy(otile.at[pl.ds(d * N, N)],
                                   out.at[row0 + d], semw)
                  for d in range(DBLK)]
    for cp in pend_w:
        cp.wait()
    scope_perm.__exit__(None, None, None)


@jax.jit
def kernel(x):
    xt = x.transpose(0, 2, 1).reshape(B * D, N)
    run = pl.kernel(
        _sc_body,
        out_type=jax.ShapeDtypeStruct((B * D, N), jnp.float32),
        mesh=plsc.VectorSubcoreMesh(core_axis_name="c", subcore_axis_name="s",
                                    num_cores=NC, num_subcores=NS),
        compiler_params=pltpu.CompilerParams(needs_layout_passes=False,
                                             use_tc_tiling_on_sc=True),
        scratch_types=[
            pltpu.VMEM((N,), jnp.int32),
            pltpu.VMEM((N,), jnp.int32),
            pltpu.VMEM((N,), jnp.int32),
            pltpu.VMEM((N,), jnp.int32),
            pltpu.VMEM((N,), jnp.int32),
            pltpu.VMEM((N,), jnp.int32),
            pltpu.VMEM((N,), jnp.int32),
            pltpu.VMEM((N,), jnp.int32),
            pltpu.VMEM((N,), jnp.int32),
            pltpu.VMEM((N,), jnp.int32),
            pltpu.VMEM((DBLK * N,), jnp.float32),
            pltpu.VMEM((DBLK * N,), jnp.float32),
            pltpu.VMEM((DBLK * N,), jnp.float32),
            pltpu.SemaphoreType.DMA,
            pltpu.SemaphoreType.DMA,
        ],
    )
    ot = run(xt)
    return ot.reshape(B, D, N).transpose(0, 2, 1)


# speedup vs baseline: 1.0071x; 1.0071x over previous
import functools

import jax
import jax.numpy as jnp
from jax import lax
from jax.experimental import pallas as pl
from jax.experimental.pallas import tpu as pltpu
from jax.experimental.pallas import tpu_sc as plsc

B = 64
N = 4096
D = 64
NC = 2
NS = 16
NW = NC * NS
L = 16
NV = N // L
HJ = NV // 2
NH = 2 * N
DBLK = 4
NBLK = D // DBLK
NGLB = 2 * NBLK


def _sc_body(xt, out, ka0, kb0, ia0, ib0, ka1, kb1, ia1, ib1, h0, h1,
             inc0, inc1, base0, base1, xtile0, xtile1, otile, sem, semw):
    lane = lax.iota(jnp.int32, L)
    ones = jnp.full((L,), 1, jnp.int32)
    lanNV = lane * NV
    lan16 = lane * L + 15
    wid = lax.axis_index("s") * NC + lax.axis_index("c")
    b0 = wid * 2
    xtiles = (xtile0, xtile1)

    cpk0 = pltpu.async_copy(xt.at[b0 * D], otile.at[pl.ds(0, N)], sem)
    cpk1 = pltpu.async_copy(xt.at[(b0 + 1) * D], otile.at[pl.ds(N, N)], sem)

    def fetch(g):
        r, blk = g // NBLK, g % NBLK
        row0 = (b0 + r) * D + blk * DBLK
        xtile = xtiles[g % 2]
        return [pltpu.async_copy(xt.at[row0 + d],
                                 xtile.at[pl.ds(d * N, N)], sem)
                for d in range(DBLK)]

    pend_f = [fetch(0), fetch(1)]
    cpk0.wait()
    cpk1.wait()

    @plsc.parallel_loop(0, NV, unroll=4)
    def init_body(i):
        for r, ka in ((0, ka0), (1, ka1)):
            v = plsc.bitcast(otile[pl.ds(r * N + i * L, L)], jnp.int32)
            ka[pl.ds(i * L, L)] = jnp.where(v < 0, ~v,
                                            v ^ jnp.int32(-2147483648))

    for p in range(4):
        if p % 2 == 0:
            pairs = ((ka0, kb0, ia0, ib0, h0), (ka1, kb1, ia1, ib1, h1))
        else:
            pairs = ((kb0, ka0, ib0, ia0, h0), (kb1, ka1, ib1, ia1, h1))
        shift = jnp.int32(8 * p)

        @plsc.parallel_loop(0, NH // L, unroll=4)
        def zero_body(i):
            h0[pl.ds(i * L, L)] = jnp.zeros((L,), jnp.int32)
            h1[pl.ds(i * L, L)] = jnp.zeros((L,), jnp.int32)

        @plsc.parallel_loop(0, HJ, unroll=2)
        def count_body(j):
            for (ks, _, _, _, h) in pairs:
                for hh in range(2):
                    pos = lanNV + (hh * HJ + j)
                    k = plsc.load_gather(ks, [pos])
                    d = lax.shift_right_logical(k, shift) & 255
                    hi = ((d * L + lane) << 1) | hh
                    plsc.addupdate_scatter(h, [hi], ones)

        @plsc.parallel_loop(0, NH // L, unroll=4)
        def scanA(i):
            for h, incb in ((h0, inc0), (h1, inc1)):
                v = h[pl.ds(i * L, L)]
                inc = plsc.cumsum(v)
                incb[pl.ds(i * L, L)] = inc
                h[pl.ds(i * L, L)] = inc - v

        def scanB(g, c):
            c0, c1 = c
            idx = g * (L * L) + lan16
            tv0 = plsc.load_gather(inc0, [idx])
            ti0 = plsc.cumsum(tv0)
            base0[pl.ds(g * L, L)] = ti0 - tv0 + c0
            tv1 = plsc.load_gather(inc1, [idx])
            ti1 = plsc.cumsum(tv1)
            base1[pl.ds(g * L, L)] = ti1 - tv1 + c1
            return (c0 + ti0[15], c1 + ti1[15])

        lax.fori_loop(0, NH // (L * L), scanB,
                      (jnp.int32(0), jnp.int32(0)))

        @plsc.parallel_loop(0, NH // (L * L), unroll=2)
        def scanC(g):
            for h, base in ((h0, base0), (h1, base1)):
                bv = base[pl.ds(g * L, L)]
                for k in range(L):
                    off = (g * L + k) * L
                    h[pl.ds(off, L)] = h[pl.ds(off, L)] + bv[k]

        def perm_body(j, _):
            for (ks, kd, is_, id_, h) in pairs:
                for hh in range(2):
                    pos = lanNV + (hh * HJ + j)
                    k = plsc.load_gather(ks, [pos])
                    if p == 0:
                        v = pos
                    else:
                        v = plsc.load_gather(is_, [pos])
                    d = lax.shift_right_logical(k, shift) & 255
                    hi = ((d * L + lane) << 1) | hh
                    dst = plsc.load_gather(h, [hi])
                    if p != 3:
                        plsc.store_scatter(kd, [dst], k)
                    plsc.store_scatter(id_, [dst], v)
                    plsc.addupdate_scatter(h, [hi], ones)
            return 0

        lax.fori_loop(0, HJ, perm_body, 0)


    pend_w = []
    for g in range(NGLB):
        r, blk = g // NBLK, g % NBLK
        row0 = (b0 + r) * D + blk * DBLK
        xtile = xtiles[g % 2]
        ia = ia0 if r == 0 else ia1
        for cp in pend_f[0]:
            cp.wait()
        pend_f = pend_f[1:]
        for cp in pend_w:
            cp.wait()

        @plsc.parallel_loop(0, NV, unroll=4)
        def permute_body(i):
            ordv = ia[pl.ds(i * L, L)]
            for d in range(DBLK):
                v = plsc.load_gather(xtile.at[pl.ds(d * N, N)], [ordv])
                otile[pl.ds(d * N + i * L, L)] = v

        if g + 2 < NGLB:
            pend_f.append(fetch(g + 2))
        pend_w = [pltpu.async_copy(otile.at[pl.ds(d * N, N)],
                                   out.at[row0 + d], semw)
                  for d in range(DBLK)]
    for cp in pend_w:
        cp.wait()


@jax.jit
def kernel(x):
    xt = x.transpose(0, 2, 1).reshape(B * D, N)
    run = pl.kernel(
        _sc_body,
        out_type=jax.ShapeDtypeStruct((B * D, N), jnp.float32),
        mesh=plsc.VectorSubcoreMesh(core_axis_name="c", subcore_axis_name="s",
                                    num_cores=NC, num_subcores=NS),
        compiler_params=pltpu.CompilerParams(needs_layout_passes=False,
                                             use_tc_tiling_on_sc=True),
        scratch_types=[
            pltpu.VMEM((N,), jnp.int32),
            pltpu.VMEM((N,), jnp.int32),
            pltpu.VMEM((N,), jnp.int32),
            pltpu.VMEM((N,), jnp.int32),
            pltpu.VMEM((N,), jnp.int32),
            pltpu.VMEM((N,), jnp.int32),
            pltpu.VMEM((N,), jnp.int32),
            pltpu.VMEM((N,), jnp.int32),
            pltpu.VMEM((NH,), jnp.int32),
            pltpu.VMEM((NH,), jnp.int32),
            pltpu.VMEM((NH,), jnp.int32),
            pltpu.VMEM((NH,), jnp.int32),
            pltpu.VMEM((NH // L,), jnp.int32),
            pltpu.VMEM((NH // L,), jnp.int32),
            pltpu.VMEM((DBLK * N,), jnp.float32),
            pltpu.VMEM((DBLK * N,), jnp.float32),
            pltpu.VMEM((DBLK * N,), jnp.float32),
            pltpu.SemaphoreType.DMA,
            pltpu.SemaphoreType.DMA,
        ],
    )
    ot = run(xt)
    return ot.reshape(B, D, N).transpose(0, 2, 1)


# speedup vs baseline: 1.0583x; 1.0509x over previous
import functools

import jax
import jax.numpy as jnp
from jax import lax
from jax.experimental import pallas as pl
from jax.experimental.pallas import tpu as pltpu
from jax.experimental.pallas import tpu_sc as plsc

B = 64
N = 4096
D = 64
NC = 2
NS = 16
NW = NC * NS
L = 16
NV = N // L
DBLK = 4
NBLK = D // DBLK
NGLB = 2 * NBLK


def _sc_body(xt, out, ka0, kb0, ia0, ib0, h0, ka1, kb1, ia1, ib1, h1,
             inc0, inc1, base0, base1, xtile0, xtile1, otile, sem, semw):
    lane = lax.iota(jnp.int32, L)
    ones = jnp.full((L,), 1, jnp.int32)
    lanNV = lane * NV
    lan16 = lane * L + 15
    wid = lax.axis_index("s") * NC + lax.axis_index("c")
    b0 = wid * 2
    xtiles = (xtile0, xtile1)

    cpk0 = pltpu.async_copy(xt.at[b0 * D], otile.at[pl.ds(0, N)], sem)
    cpk1 = pltpu.async_copy(xt.at[(b0 + 1) * D], otile.at[pl.ds(N, N)], sem)

    def fetch(g):
        r, blk = g // NBLK, g % NBLK
        row0 = (b0 + r) * D + blk * DBLK
        xtile = xtiles[g % 2]
        return [pltpu.async_copy(xt.at[row0 + d],
                                 xtile.at[pl.ds(d * N, N)], sem)
                for d in range(DBLK)]

    pend_f = [fetch(0), fetch(1)]
    cpk0.wait()
    cpk1.wait()

    @plsc.parallel_loop(0, NV, unroll=4)
    def init_body(i):
        for r, ka in ((0, ka0), (1, ka1)):
            v = plsc.bitcast(otile[pl.ds(r * N + i * L, L)], jnp.int32)
            ka[pl.ds(i * L, L)] = jnp.where(v < 0, ~v,
                                            v ^ jnp.int32(-2147483648))

    for p in range(4):
        if p % 2 == 0:
            pairs = ((ka0, kb0, ia0, ib0, h0), (ka1, kb1, ia1, ib1, h1))
        else:
            pairs = ((kb0, ka0, ib0, ia0, h0), (kb1, ka1, ib1, ia1, h1))
        shift = jnp.int32(8 * p)

        @plsc.parallel_loop(0, NV, unroll=4)
        def zero_body(i):
            h0[pl.ds(i * L, L)] = jnp.zeros((L,), jnp.int32)
            h1[pl.ds(i * L, L)] = jnp.zeros((L,), jnp.int32)

        @plsc.parallel_loop(0, NV, unroll=4)
        def count_body(i):
            pos = lanNV + i
            for (ks, _, _, _, h) in pairs:
                k = plsc.load_gather(ks, [pos])
                d = lax.shift_right_logical(k, shift) & 255
                plsc.addupdate_scatter(h, [d * L + lane], ones)

        @plsc.parallel_loop(0, NV, unroll=4)
        def scanA(i):
            for h, incb in ((h0, inc0), (h1, inc1)):
                v = h[pl.ds(i * L, L)]
                inc = plsc.cumsum(v)
                incb[pl.ds(i * L, L)] = inc
                h[pl.ds(i * L, L)] = inc - v

        def scanB(g, c):
            c0, c1 = c
            idx = g * (L * L) + lan16
            tv0 = plsc.load_gather(inc0, [idx])
            ti0 = plsc.cumsum(tv0)
            base0[pl.ds(g * L, L)] = ti0 - tv0 + c0
            tv1 = plsc.load_gather(inc1, [idx])
            ti1 = plsc.cumsum(tv1)
            base1[pl.ds(g * L, L)] = ti1 - tv1 + c1
            return (c0 + ti0[15], c1 + ti1[15])

        lax.fori_loop(0, NV // L, scanB, (jnp.int32(0), jnp.int32(0)))

        @plsc.parallel_loop(0, NV // L, unroll=2)
        def scanC(g):
            for h, base in ((h0, base0), (h1, base1)):
                bv = base[pl.ds(g * L, L)]
                for k in range(L):
                    off = (g * L + k) * L
                    h[pl.ds(off, L)] = h[pl.ds(off, L)] + bv[k]

        def perm_body(i, _):
            pos = lanNV + i
            for (ks, kd, is_, id_, h) in pairs:
                k = plsc.load_gather(ks, [pos])
                if p == 0:
                    v = pos
                else:
                    v = plsc.load_gather(is_, [pos])
                d = lax.shift_right_logical(k, shift) & 255
                hi = d * L + lane
                dst = plsc.load_gather(h, [hi])
                if p != 3:
                    plsc.store_scatter(kd, [dst], k)
                plsc.store_scatter(id_, [dst], v)
                plsc.addupdate_scatter(h, [hi], ones)
            return 0

        lax.fori_loop(0, NV, perm_body, 0)


    pend_w = []
    for g in range(NGLB):
        r, blk = g // NBLK, g % NBLK
        row0 = (b0 + r) * D + blk * DBLK
        xtile = xtiles[g % 2]
        ia = ia0 if r == 0 else ia1
        for cp in pend_f[0]:
            cp.wait()
        pend_f = pend_f[1:]
        for cp in pend_w:
            cp.wait()

        @plsc.parallel_loop(0, NV, unroll=4)
        def permute_body(i):
            ordv = ia[pl.ds(i * L, L)]
            for d in range(DBLK):
                v = plsc.load_gather(xtile.at[pl.ds(d * N, N)], [ordv])
                otile[pl.ds(d * N + i * L, L)] = v

        if g + 2 < NGLB:
            pend_f.append(fetch(g + 2))
        pend_w = [pltpu.async_copy(otile.at[pl.ds(d * N, N)],
                                   out.at[row0 + d], semw)
                  for d in range(DBLK)]
    for cp in pend_w:
        cp.wait()


@jax.jit
def kernel(x):
    xt = x.transpose(0, 2, 1).reshape(B * D, N)
    run = pl.kernel(
        _sc_body,
        out_type=jax.ShapeDtypeStruct((B * D, N), jnp.float32),
        mesh=plsc.VectorSubcoreMesh(core_axis_name="c", subcore_axis_name="s",
                                    num_cores=NC, num_subcores=NS),
        compiler_params=pltpu.CompilerParams(needs_layout_passes=False,
                                             use_tc_tiling_on_sc=True),
        scratch_types=[
            pltpu.VMEM((N,), jnp.int32),
            pltpu.VMEM((N,), jnp.int32),
            pltpu.VMEM((N,), jnp.int32),
            pltpu.VMEM((N,), jnp.int32),
            pltpu.VMEM((N,), jnp.int32),
            pltpu.VMEM((N,), jnp.int32),
            pltpu.VMEM((N,), jnp.int32),
            pltpu.VMEM((N,), jnp.int32),
            pltpu.VMEM((N,), jnp.int32),
            pltpu.VMEM((N,), jnp.int32),
            pltpu.VMEM((N,), jnp.int32),
            pltpu.VMEM((N,), jnp.int32),
            pltpu.VMEM((NV,), jnp.int32),
            pltpu.VMEM((NV,), jnp.int32),
            pltpu.VMEM((DBLK * N,), jnp.float32),
            pltpu.VMEM((DBLK * N,), jnp.float32),
            pltpu.VMEM((DBLK * N,), jnp.float32),
            pltpu.SemaphoreType.DMA,
            pltpu.SemaphoreType.DMA,
        ],
    )
    ot = run(xt)
    return ot.reshape(B, D, N).transpose(0, 2, 1)
